# baseline (speedup 1.0000x reference)
import functools

import jax
import jax.numpy as jnp
from jax import lax
from jax.experimental import pallas as pl
from jax.experimental.pallas import tpu as pltpu
from jax.experimental.pallas import tpu_sc as plsc

B, L, D = 4096, 50, 128
T = B * L
NC, NS = 2, 16
NW = NC * NS
CH = 128
SLICES = (40960,) * 5
TB = 4096


@functools.cache
def _get_sc_gather(ts):
    tpw = ts // NW
    nchunk = tpw // CH
    assert nchunk % 2 == 0 and tpw % CH == 0
    mesh = plsc.VectorSubcoreMesh(core_axis_name="c", subcore_axis_name="s")

    @functools.partial(
        pl.kernel,
        out_type=[
            jax.ShapeDtypeStruct((ts // 2, D), jnp.float32),
            jax.ShapeDtypeStruct((ts // 2, D), jnp.float32),
        ],
        mesh=mesh,
        scratch_types=(
            [pltpu.VMEM((CH,), jnp.int32)] * 4
            + [pltpu.VMEM((CH, D), jnp.float32)] * 4
            + [pltpu.SemaphoreType.DMA] * 12
        ),
    )
    def _sc_gather(user_table, item_table, uidx, iidx, out_u, out_i,
                   ui0, ui1, ii0, ii1, ur0, ur1, ir0, ir1,
                   uis0, uis1, iis0, iis1, ugs0, ugs1, igs0, igs1,
                   uos0, uos1, ios0, ios1):
        wid = lax.axis_index("s") * NC + lax.axis_index("c")
        base = wid * tpw

        U = (user_table, uidx, out_u, (ui0, ui1), (ur0, ur1),
             (uis0, uis1), (ugs0, ugs1), (uos0, uos1))
        I = (item_table, iidx, out_i, (ii0, ii1), (ir0, ir1),
             (iis0, iis1), (igs0, igs1), (ios0, ios1))

        def make_ops(t):
            tbl, idx_hbm, out_hbm, idxs, rows, isems, gsems, osems = t

            def idx_start(p, c):
                off = jnp.minimum(base + c * CH, ts - CH)
                pltpu.async_copy(idx_hbm.at[pl.ds(off, CH)], idxs[p],
                                 isems[p])

            def idx_wait(p):
                pltpu.make_async_copy(idx_hbm.at[pl.ds(0, CH)], idxs[p],
                                      isems[p]).wait()

            def g_start(p):
                pltpu.async_copy(tbl.at[idxs[p]], rows[p], gsems[p])

            def g_wait(p):
                pltpu.make_async_copy(tbl.at[idxs[p]], rows[p],
                                      gsems[p]).wait()

            def convert(p):
                rf = rows[p]

                def crow(r2, carry):
                    for j in range(8):
                        a = rf[2 * r2, pl.ds(j * 16, 16)]
                        b = rf[2 * r2 + 1, pl.ds(j * 16, 16)]
                        ai = lax.bitcast_convert_type(a, jnp.int32)
                        bi = lax.bitcast_convert_type(b, jnp.int32)
                        w = ((bi & jnp.int32(-65536))
                             | lax.shift_right_logical(ai, 16))
                        rf[r2, pl.ds(j * 16, 16)] = (
                            lax.bitcast_convert_type(w, jnp.float32))
                    return carry

                lax.fori_loop(0, CH // 2, crow, 0)

            def o_start(p, c):
                off = pl.multiple_of((base + c * CH) // 2, CH // 2)
                pltpu.async_copy(rows[p].at[pl.ds(0, CH // 2)],
                                 out_hbm.at[pl.ds(off, CH // 2)], osems[p])

            def o_wait(p):
                pltpu.make_async_copy(rows[p].at[pl.ds(0, CH // 2)],
                                      out_hbm.at[pl.ds(0, CH // 2)],
                                      osems[p]).wait()

            return (idx_start, idx_wait, g_start, g_wait, convert,
                    o_start, o_wait)

        u_ops = make_ops(U)
        i_ops = make_ops(I)
        both = (u_ops, i_ops)

        for idx_start, idx_wait, g_start, g_wait, cv, o_start, o_wait in both:
            idx_start(0, 0)
            idx_start(1, 1)
        for idx_start, idx_wait, g_start, g_wait, cv, o_start, o_wait in both:
            idx_wait(0)
            g_start(0)

        def body(i, carry):
            c = 2 * i
            for (idx_start, idx_wait, g_start, g_wait, cv, o_start,
                 o_wait) in both:
                idx_wait(1)
                g_start(1)
            for (idx_start, idx_wait, g_start, g_wait, cv, o_start,
                 o_wait) in both:
                g_wait(0)
                cv(0)
                o_start(0, c)
                idx_start(0, c + 2)
            for (idx_start, idx_wait, g_start, g_wait, cv, o_start,
                 o_wait) in both:
                g_wait(1)
                cv(1)
                o_start(1, c + 1)
                idx_start(1, c + 3)
            for (idx_start, idx_wait, g_start, g_wait, cv, o_start,
                 o_wait) in both:
                idx_wait(0)
                g_start(0)
            for (idx_start, idx_wait, g_start, g_wait, cv, o_start,
                 o_wait) in both:
                o_wait(0)
                o_wait(1)
            return carry

        lax.fori_loop(0, nchunk // 2, body, 0)
        for idx_start, idx_wait, g_start, g_wait, cv, o_start, o_wait in both:
            g_wait(0)
            idx_wait(1)

    return _sc_gather


_DN = (((1,), (1,)), ((), ()))


def _mlp_body(u_ref, i_ref, w1u_ref, w1i_ref, b1_ref, w2_ref, b2_ref,
              w3_ref, b3_ref, w4_ref, b4_ref, out_ref):
    u = pltpu.bitcast(u_ref[...], jnp.bfloat16)
    it = pltpu.bitcast(i_ref[...], jnp.bfloat16)
    h = lax.dot_general(w1u_ref[...], u, _DN,
                        preferred_element_type=jnp.float32)
    h = h + lax.dot_general(w1i_ref[...], it, _DN,
                            preferred_element_type=jnp.float32)
    h = jax.nn.relu(h + b1_ref[...])
    h = jnp.dot(w2_ref[...], h.astype(jnp.bfloat16),
                preferred_element_type=jnp.float32)
    h = jax.nn.relu(h + b2_ref[...])
    h = jnp.dot(w3_ref[...], h.astype(jnp.bfloat16),
                preferred_element_type=jnp.float32)
    h = jax.nn.relu(h + b3_ref[...])
    lg = jnp.dot(w4_ref[...], h.astype(jnp.bfloat16),
                 preferred_element_type=jnp.float32)
    lg = lg[0:1] + b4_ref[0, 0]
    out_ref[...] = jax.nn.sigmoid(lg).reshape(1, 1, TB)


def _mk_mlp_specs(ts):
    return dict(
        in_specs=[
            pl.BlockSpec((TB // 2, D), lambda g: (g, 0)),
            pl.BlockSpec((TB // 2, D), lambda g: (g, 0)),
            pl.BlockSpec((256, D), lambda g: (0, 0)),
            pl.BlockSpec((256, D), lambda g: (0, 0)),
            pl.BlockSpec((256, 1), lambda g: (0, 0)),
            pl.BlockSpec((D, 256), lambda g: (0, 0)),
            pl.BlockSpec((D, 1), lambda g: (0, 0)),
            pl.BlockSpec((64, D), lambda g: (0, 0)),
            pl.BlockSpec((64, 1), lambda g: (0, 0)),
            pl.BlockSpec((8, 64), lambda g: (0, 0)),
            pl.BlockSpec(memory_space=pltpu.SMEM),
        ],
        out_specs=pl.BlockSpec((1, 1, TB), lambda g: (g, 0, 0)),
        out_shape=jax.ShapeDtypeStruct((ts // TB, 1, TB), jnp.float32),
    )


@functools.cache
def _get_mlp(ts):
    return pl.pallas_call(_mlp_body, grid=(ts // TB,), **_mk_mlp_specs(ts))


def kernel(user_matrix, item_matrix, user_table, item_table,
           W1, b1, W2, b2, W3, b3, W4, b4):
    uidx = user_matrix.reshape(-1).astype(jnp.int32)
    iidx = item_matrix.reshape(-1).astype(jnp.int32)

    w1b = W1.astype(jnp.bfloat16)
    w1u = w1b[:, :D]
    w1i = w1b[:, D:]
    w2b = W2.astype(jnp.bfloat16)
    w3b = W3.astype(jnp.bfloat16)
    w4b = jnp.broadcast_to(W4, (8, 64)).astype(jnp.bfloat16)
    wargs = (w1u, w1i, b1.reshape(256, 1), w2b, b2.reshape(D, 1),
             w3b, b3.reshape(64, 1), w4b, b4.reshape(1, 1))

    outs = []
    off = 0
    for ts in SLICES:
        u_rows, i_rows = _get_sc_gather(ts)(user_table, item_table,
                                            uidx[off:off + ts],
                                            iidx[off:off + ts])
        outs.append(_get_mlp(ts)(u_rows, i_rows, *wargs).reshape(ts))
        off += ts
    return jnp.concatenate(outs).reshape(B, L)


# speedup vs baseline: 1.5852x; 1.5852x over previous
import functools

import jax
import jax.numpy as jnp
from jax import lax
from jax.experimental import pallas as pl
from jax.experimental.pallas import tpu as pltpu
from jax.experimental.pallas import tpu_sc as plsc

B, L, D = 4096, 50, 128
T = B * L
NC, NS = 2, 16
NW = NC * NS
CH = 128
SLICES = (40960,) * 5
TB = 4096


@functools.cache
def _get_sc_gather(ts):
    tpw = ts // NW
    nchunk = tpw // CH
    assert nchunk % 2 == 0 and tpw % CH == 0
    mesh = plsc.VectorSubcoreMesh(core_axis_name="c", subcore_axis_name="s")

    @functools.partial(
        pl.kernel,
        out_type=[
            jax.ShapeDtypeStruct((ts, D), jnp.float32),
            jax.ShapeDtypeStruct((ts, D), jnp.float32),
        ],
        mesh=mesh,
        scratch_types=(
            [pltpu.VMEM((CH,), jnp.int32)] * 4
            + [pltpu.VMEM((CH, D), jnp.float32)] * 4
            + [pltpu.SemaphoreType.DMA] * 12
        ),
    )
    def _sc_gather(user_table, item_table, uidx, iidx, out_u, out_i,
                   ui0, ui1, ii0, ii1, ur0, ur1, ir0, ir1,
                   uis0, uis1, iis0, iis1, ugs0, ugs1, igs0, igs1,
                   uos0, uos1, ios0, ios1):
        wid = lax.axis_index("s") * NC + lax.axis_index("c")
        base = wid * tpw

        U = (user_table, uidx, out_u, (ui0, ui1), (ur0, ur1),
             (uis0, uis1), (ugs0, ugs1), (uos0, uos1))
        I = (item_table, iidx, out_i, (ii0, ii1), (ir0, ir1),
             (iis0, iis1), (igs0, igs1), (ios0, ios1))

        def make_ops(t):
            tbl, idx_hbm, out_hbm, idxs, rows, isems, gsems, osems = t

            def idx_start(p, c):
                off = jnp.minimum(base + c * CH, ts - CH)
                pltpu.async_copy(idx_hbm.at[pl.ds(off, CH)], idxs[p],
                                 isems[p])

            def idx_wait(p):
                pltpu.make_async_copy(idx_hbm.at[pl.ds(0, CH)], idxs[p],
                                      isems[p]).wait()

            def g_start(p):
                pltpu.async_copy(tbl.at[idxs[p]], rows[p], gsems[p])

            def g_wait(p):
                pltpu.make_async_copy(tbl.at[idxs[p]], rows[p],
                                      gsems[p]).wait()

            def o_start(p, c):
                off = pl.multiple_of(base + c * CH, CH)
                pltpu.async_copy(rows[p], out_hbm.at[pl.ds(off, CH)],
                                 osems[p])

            def o_wait(p):
                pltpu.make_async_copy(rows[p], out_hbm.at[pl.ds(0, CH)],
                                      osems[p]).wait()

            return (idx_start, idx_wait, g_start, g_wait, o_start, o_wait)

        u_ops = make_ops(U)
        i_ops = make_ops(I)
        both = (u_ops, i_ops)

        for idx_start, idx_wait, g_start, g_wait, o_start, o_wait in both:
            idx_start(0, 0)
            idx_start(1, 1)
        for idx_start, idx_wait, g_start, g_wait, o_start, o_wait in both:
            idx_wait(0)
            g_start(0)

        def body(i, carry):
            c = 2 * i
            for (idx_start, idx_wait, g_start, g_wait, o_start,
                 o_wait) in both:
                idx_wait(1)
                g_start(1)
            for (idx_start, idx_wait, g_start, g_wait, o_start,
                 o_wait) in both:
                g_wait(0)
                o_start(0, c)
                idx_start(0, c + 2)
            for (idx_start, idx_wait, g_start, g_wait, o_start,
                 o_wait) in both:
                g_wait(1)
                o_start(1, c + 1)
                idx_start(1, c + 3)
            for (idx_start, idx_wait, g_start, g_wait, o_start,
                 o_wait) in both:
                idx_wait(0)
                o_wait(0)
                g_start(0)
            for (idx_start, idx_wait, g_start, g_wait, o_start,
                 o_wait) in both:
                o_wait(1)
            return carry

        lax.fori_loop(0, nchunk // 2, body, 0)
        for idx_start, idx_wait, g_start, g_wait, o_start, o_wait in both:
            g_wait(0)
            idx_wait(1)

    return _sc_gather


_DN = (((1,), (1,)), ((), ()))


def _mlp_body(u_ref, i_ref, w1u_ref, w1i_ref, b1_ref, w2_ref, b2_ref,
              w3_ref, b3_ref, w4_ref, b4_ref, out_ref):
    u = u_ref[...].astype(jnp.bfloat16)
    it = i_ref[...].astype(jnp.bfloat16)
    h = lax.dot_general(w1u_ref[...], u, _DN,
                        preferred_element_type=jnp.float32)
    h = h + lax.dot_general(w1i_ref[...], it, _DN,
                            preferred_element_type=jnp.float32)
    h = jax.nn.relu(h + b1_ref[...])
    h = jnp.dot(w2_ref[...], h.astype(jnp.bfloat16),
                preferred_element_type=jnp.float32)
    h = jax.nn.relu(h + b2_ref[...])
    h = jnp.dot(w3_ref[...], h.astype(jnp.bfloat16),
                preferred_element_type=jnp.float32)
    h = jax.nn.relu(h + b3_ref[...])
    lg = jnp.dot(w4_ref[...], h.astype(jnp.bfloat16),
                 preferred_element_type=jnp.float32)
    lg = lg[0:1] + b4_ref[0, 0]
    out_ref[...] = jax.nn.sigmoid(lg).reshape(1, 1, TB)


def _mk_mlp_specs(ts):
    return dict(
        in_specs=[
            pl.BlockSpec((TB, D), lambda g: (g, 0)),
            pl.BlockSpec((TB, D), lambda g: (g, 0)),
            pl.BlockSpec((256, D), lambda g: (0, 0)),
            pl.BlockSpec((256, D), lambda g: (0, 0)),
            pl.BlockSpec((256, 1), lambda g: (0, 0)),
            pl.BlockSpec((D, 256), lambda g: (0, 0)),
            pl.BlockSpec((D, 1), lambda g: (0, 0)),
            pl.BlockSpec((64, D), lambda g: (0, 0)),
            pl.BlockSpec((64, 1), lambda g: (0, 0)),
            pl.BlockSpec((8, 64), lambda g: (0, 0)),
            pl.BlockSpec(memory_space=pltpu.SMEM),
        ],
        out_specs=pl.BlockSpec((1, 1, TB), lambda g: (g, 0, 0)),
        out_shape=jax.ShapeDtypeStruct((ts // TB, 1, TB), jnp.float32),
    )


@functools.cache
def _get_mlp(ts):
    return pl.pallas_call(_mlp_body, grid=(ts // TB,), **_mk_mlp_specs(ts))


def kernel(user_matrix, item_matrix, user_table, item_table,
           W1, b1, W2, b2, W3, b3, W4, b4):
    uidx = user_matrix.reshape(-1).astype(jnp.int32)
    iidx = item_matrix.reshape(-1).astype(jnp.int32)

    w1b = W1.astype(jnp.bfloat16)
    w1u = w1b[:, :D]
    w1i = w1b[:, D:]
    w2b = W2.astype(jnp.bfloat16)
    w3b = W3.astype(jnp.bfloat16)
    w4b = jnp.broadcast_to(W4, (8, 64)).astype(jnp.bfloat16)
    wargs = (w1u, w1i, b1.reshape(256, 1), w2b, b2.reshape(D, 1),
             w3b, b3.reshape(64, 1), w4b, b4.reshape(1, 1))

    outs = []
    off = 0
    for ts in SLICES:
        u_rows, i_rows = _get_sc_gather(ts)(user_table, item_table,
                                            uidx[off:off + ts],
                                            iidx[off:off + ts])
        outs.append(_get_mlp(ts)(u_rows, i_rows, *wargs).reshape(ts))
        off += ts
    return jnp.concatenate(outs).reshape(B, L)
